# baseline (speedup 1.0000x reference)
import functools

import jax
import jax.numpy as jnp
from jax import lax
from jax.experimental import pallas as pl
from jax.experimental.pallas import tpu as pltpu
from jax.experimental.pallas import tpu_sc as plsc

_NUM_CORES = 2
_NUM_SUBCORES = 16
_NUM_WORKERS = _NUM_CORES * _NUM_SUBCORES
_LANES = 16
_DEPTH = 4


def _mf_forward(users, items, user_embeddings, item_embeddings,
                user_biases, item_biases):
    batch = users.shape[0]
    d = user_embeddings.shape[1]
    assert batch % (_NUM_WORKERS * _LANES) == 0
    bpw = batch // _NUM_WORKERS
    n_blocks = bpw // _LANES
    stage = _DEPTH * _LANES
    mesh = plsc.VectorSubcoreMesh(
        core_axis_name="c", subcore_axis_name="s", num_cores=_NUM_CORES)

    @functools.partial(
        pl.kernel,
        mesh=mesh,
        compiler_params=pltpu.CompilerParams(needs_layout_passes=False,
                                             use_tc_tiling_on_sc=True),
        out_type=jax.ShapeDtypeStruct((batch,), jnp.float32),
        scratch_types=[
            pltpu.VMEM((bpw,), jnp.int32),
            pltpu.VMEM((bpw,), jnp.int32),
            pltpu.VMEM((stage, d), jnp.float32),
            pltpu.VMEM((stage, d), jnp.float32),
            pltpu.VMEM((stage, 1), jnp.float32),
            pltpu.VMEM((stage, 1), jnp.float32),
            pltpu.VMEM((bpw,), jnp.float32),
            pltpu.SemaphoreType.DMA,
            pltpu.SemaphoreType.DMA,
            pltpu.SemaphoreType.DMA,
            pltpu.SemaphoreType.DMA,
        ],
    )
    def mf_kernel(users_hbm, items_hbm, ue_hbm, ie_hbm, ub_hbm, ib_hbm,
                  out_hbm, uidx_v, iidx_v, du_v, di_v, dub_v, dib_v,
                  out_v, sem_u, sem_i, sem_ub, sem_ib):
        wid = lax.axis_index("s") * _NUM_CORES + lax.axis_index("c")
        base = wid * bpw

        pltpu.sync_copy(users_hbm.at[pl.ds(base, bpw)], uidx_v)
        pltpu.sync_copy(items_hbm.at[pl.ds(base, bpw)], iidx_v)

        lane = lax.iota(jnp.int32, _LANES)
        zero = jnp.zeros((_LANES,), jnp.int32)

        def fire_block(g):
            slot = lax.rem(g, _DEPTH) * _LANES
            ub16 = uidx_v[pl.ds(g * _LANES, _LANES)]
            ib16 = iidx_v[pl.ds(g * _LANES, _LANES)]
            for k in range(_LANES):
                pltpu.async_copy(ue_hbm.at[ub16[k]], du_v.at[slot + k],
                                 sem_u)
                pltpu.async_copy(ie_hbm.at[ib16[k]], di_v.at[slot + k],
                                 sem_i)
                pltpu.async_copy(ub_hbm.at[ub16[k]], dub_v.at[slot + k],
                                 sem_ub)
                pltpu.async_copy(ib_hbm.at[ib16[k]], dib_v.at[slot + k],
                                 sem_ib)

        def drain_block():
            pltpu.make_async_copy(
                ue_hbm.at[pl.ds(0, _LANES)],
                du_v.at[pl.ds(0, _LANES)], sem_u).wait()
            pltpu.make_async_copy(
                ie_hbm.at[pl.ds(0, _LANES)],
                di_v.at[pl.ds(0, _LANES)], sem_i).wait()
            pltpu.make_async_copy(
                ub_hbm.at[pl.ds(0, _LANES)],
                dub_v.at[pl.ds(0, _LANES)], sem_ub).wait()
            pltpu.make_async_copy(
                ib_hbm.at[pl.ds(0, _LANES)],
                dib_v.at[pl.ds(0, _LANES)], sem_ib).wait()

        def compute_block(g):
            slot = lax.rem(g, _DEPTH) * _LANES
            rows = slot + lane
            acc = (plsc.load_gather(dub_v, [rows, zero])
                   + plsc.load_gather(dib_v, [rows, zero]))
            for c in range(d):
                cc = jnp.full((_LANES,), c, jnp.int32)
                acc = acc + (plsc.load_gather(du_v, [rows, cc])
                             * plsc.load_gather(di_v, [rows, cc]))
            out_v[pl.ds(g * _LANES, _LANES)] = acc

        def step(g, _):
            @pl.when(g < n_blocks)
            def _fire():
                fire_block(g)

            @pl.when(g >= _DEPTH - 1)
            def _consume():
                drain_block()
                compute_block(g - (_DEPTH - 1))

            return _

        lax.fori_loop(0, n_blocks + _DEPTH - 1, step, None)

        pltpu.sync_copy(out_v, out_hbm.at[pl.ds(base, bpw)])

    return mf_kernel(users, items, user_embeddings, item_embeddings,
                     user_biases, item_biases)


def kernel(users, items, user_embeddings, item_embeddings, user_biases,
           item_biases):
    out = _mf_forward(users.astype(jnp.int32), items.astype(jnp.int32),
                      user_embeddings, item_embeddings,
                      user_biases, item_biases)
    return out.reshape(-1, 1)


# speedup vs baseline: 1.0021x; 1.0021x over previous
import functools

import jax
import jax.numpy as jnp
from jax import lax
from jax.experimental import pallas as pl
from jax.experimental.pallas import tpu as pltpu
from jax.experimental.pallas import tpu_sc as plsc

_NUM_CORES = 2
_NUM_SUBCORES = 16
_NUM_WORKERS = _NUM_CORES * _NUM_SUBCORES
_LANES = 16
_DEPTH = 4


def _mf_forward(users, items, user_embeddings, item_embeddings,
                user_biases, item_biases):
    batch = users.shape[0]
    d = user_embeddings.shape[1]
    assert batch % (_NUM_WORKERS * _LANES) == 0
    bpw = batch // _NUM_WORKERS
    n_blocks = bpw // _LANES
    stage = _DEPTH * _LANES
    mesh = plsc.VectorSubcoreMesh(
        core_axis_name="c", subcore_axis_name="s", num_cores=_NUM_CORES)

    @functools.partial(
        pl.kernel,
        mesh=mesh,
        compiler_params=pltpu.CompilerParams(needs_layout_passes=False,
                                             use_tc_tiling_on_sc=True,
                                             skip_device_barrier=True,
                                             disable_semaphore_checks=True),
        out_type=jax.ShapeDtypeStruct((batch,), jnp.float32),
        scratch_types=[
            pltpu.VMEM((bpw,), jnp.int32),
            pltpu.VMEM((bpw,), jnp.int32),
            pltpu.VMEM((stage, d), jnp.float32),
            pltpu.VMEM((stage, d), jnp.float32),
            pltpu.VMEM((stage, 1), jnp.float32),
            pltpu.VMEM((stage, 1), jnp.float32),
            pltpu.VMEM((bpw,), jnp.float32),
            pltpu.SemaphoreType.DMA,
            pltpu.SemaphoreType.DMA,
            pltpu.SemaphoreType.DMA,
            pltpu.SemaphoreType.DMA,
        ],
    )
    def mf_kernel(users_hbm, items_hbm, ue_hbm, ie_hbm, ub_hbm, ib_hbm,
                  out_hbm, uidx_v, iidx_v, du_v, di_v, bu_v, bi_v,
                  out_v, sem_u, sem_i, sem_ub, sem_ib):
        wid = lax.axis_index("s") * _NUM_CORES + lax.axis_index("c")
        base = wid * bpw

        pltpu.sync_copy(users_hbm.at[pl.ds(base, bpw)], uidx_v)
        pltpu.sync_copy(items_hbm.at[pl.ds(base, bpw)], iidx_v)

        lane = lax.iota(jnp.int32, _LANES)
        zero = jnp.zeros((_LANES,), jnp.int32)

        def fire_block(g):
            slot = lax.rem(g, _DEPTH) * _LANES
            ub16 = uidx_v[pl.ds(g * _LANES, _LANES)]
            ib16 = iidx_v[pl.ds(g * _LANES, _LANES)]
            for k in range(_LANES):
                pltpu.async_copy(ue_hbm.at[ub16[k]], du_v.at[slot + k],
                                 sem_u)
                pltpu.async_copy(ie_hbm.at[ib16[k]], di_v.at[slot + k],
                                 sem_i)
                pltpu.async_copy(ub_hbm.at[ub16[k]], bu_v.at[slot + k],
                                 sem_ub)
                pltpu.async_copy(ib_hbm.at[ib16[k]], bi_v.at[slot + k],
                                 sem_ib)

        def drain_block():
            pltpu.make_async_copy(
                ue_hbm.at[pl.ds(0, _LANES)],
                du_v.at[pl.ds(0, _LANES)], sem_u).wait()
            pltpu.make_async_copy(
                ie_hbm.at[pl.ds(0, _LANES)],
                di_v.at[pl.ds(0, _LANES)], sem_i).wait()
            pltpu.make_async_copy(
                ub_hbm.at[pl.ds(0, _LANES)],
                bu_v.at[pl.ds(0, _LANES)], sem_ub).wait()
            pltpu.make_async_copy(
                ib_hbm.at[pl.ds(0, _LANES)],
                bi_v.at[pl.ds(0, _LANES)], sem_ib).wait()

        def compute_block(g):
            slot = lax.rem(g, _DEPTH) * _LANES
            rows = slot + lane
            acc = (plsc.load_gather(bu_v, [rows, zero])
                   + plsc.load_gather(bi_v, [rows, zero]))
            for c in range(d):
                cc = jnp.full((_LANES,), c, jnp.int32)
                acc = acc + (plsc.load_gather(du_v, [rows, cc])
                             * plsc.load_gather(di_v, [rows, cc]))
            out_v[pl.ds(g * _LANES, _LANES)] = acc

        def step(g, _):
            @pl.when(g < n_blocks)
            def _fire():
                fire_block(g)

            @pl.when(g >= _DEPTH - 1)
            def _consume():
                drain_block()
                compute_block(g - (_DEPTH - 1))

            return _

        lax.fori_loop(0, n_blocks + _DEPTH - 1, step, None)

        pltpu.sync_copy(out_v, out_hbm.at[pl.ds(base, bpw)])

    return mf_kernel(users, items, user_embeddings, item_embeddings,
                     user_biases, item_biases)


def kernel(users, items, user_embeddings, item_embeddings, user_biases,
           item_biases):
    out = _mf_forward(users.astype(jnp.int32), items.astype(jnp.int32),
                      user_embeddings, item_embeddings,
                      user_biases, item_biases)
    return out.reshape(-1, 1)


# speedup vs baseline: 1.4034x; 1.4005x over previous
import functools

import jax
import jax.numpy as jnp
from jax import lax
from jax.experimental import pallas as pl
from jax.experimental.pallas import tpu as pltpu
from jax.experimental.pallas import tpu_sc as plsc

_NUM_CORES = 2
_NUM_SUBCORES = 16
_NUM_WORKERS = _NUM_CORES * _NUM_SUBCORES
_LANES = 16
_DEPTH = 4


def _mf_forward(users, items, user_embeddings, item_embeddings,
                user_biases, item_biases):
    batch = users.shape[0]
    d = user_embeddings.shape[1]
    assert batch % (_NUM_WORKERS * _LANES) == 0
    bpw = batch // _NUM_WORKERS
    n_blocks = bpw // _LANES
    mesh = plsc.VectorSubcoreMesh(
        core_axis_name="c", subcore_axis_name="s", num_cores=_NUM_CORES)

    @functools.partial(
        pl.kernel,
        mesh=mesh,
        compiler_params=pltpu.CompilerParams(needs_layout_passes=False,
                                             use_tc_tiling_on_sc=True,
                                             skip_device_barrier=True,
                                             disable_semaphore_checks=True),
        out_type=jax.ShapeDtypeStruct((batch,), jnp.float32),
        scratch_types=[
            pltpu.VMEM((bpw,), jnp.int32),
            pltpu.VMEM((bpw,), jnp.int32),
            pltpu.VMEM((_DEPTH * _LANES, d), jnp.float32),
            pltpu.VMEM((_DEPTH * _LANES, d), jnp.float32),
            pltpu.VMEM((bpw,), jnp.float32),
            pltpu.VMEM((bpw,), jnp.float32),
            pltpu.VMEM((bpw,), jnp.float32),
            pltpu.SemaphoreType.DMA,
            pltpu.SemaphoreType.DMA,
            pltpu.SemaphoreType.DMA,
            pltpu.SemaphoreType.DMA,
        ],
    )
    def mf_kernel(users_hbm, items_hbm, ue_hbm, ie_hbm, ub_hbm, ib_hbm,
                  out_hbm, uidx_v, iidx_v, du_v, di_v, ubias_v, ibias_v,
                  out_v, sem_u, sem_i, sem_ub, sem_ib):
        wid = lax.axis_index("s") * _NUM_CORES + lax.axis_index("c")
        base = wid * bpw

        pltpu.sync_copy(users_hbm.at[pl.ds(base, bpw)], uidx_v)
        pltpu.sync_copy(items_hbm.at[pl.ds(base, bpw)], iidx_v)

        cp_ub = pltpu.async_copy(ub_hbm.at[uidx_v], ubias_v, sem_ub)
        cp_ib = pltpu.async_copy(ib_hbm.at[iidx_v], ibias_v, sem_ib)
        cp_ub.wait()
        cp_ib.wait()

        lane = lax.iota(jnp.int32, _LANES)

        def fire_block(g):
            slot = lax.rem(g, _DEPTH) * _LANES
            ub16 = uidx_v[pl.ds(g * _LANES, _LANES)]
            ib16 = iidx_v[pl.ds(g * _LANES, _LANES)]
            for k in range(_LANES):
                pltpu.async_copy(ue_hbm.at[ub16[k]], du_v.at[slot + k],
                                 sem_u)
                pltpu.async_copy(ie_hbm.at[ib16[k]], di_v.at[slot + k],
                                 sem_i)

        def drain_block():
            pltpu.make_async_copy(
                ue_hbm.at[pl.ds(0, _LANES)],
                du_v.at[pl.ds(0, _LANES)], sem_u).wait()
            pltpu.make_async_copy(
                ie_hbm.at[pl.ds(0, _LANES)],
                di_v.at[pl.ds(0, _LANES)], sem_i).wait()

        def compute_block(g):
            slot = lax.rem(g, _DEPTH) * _LANES
            rows = slot + lane
            acc = (ubias_v[pl.ds(g * _LANES, _LANES)]
                   + ibias_v[pl.ds(g * _LANES, _LANES)])
            for c in range(d):
                cc = jnp.full((_LANES,), c, jnp.int32)
                acc = acc + (plsc.load_gather(du_v, [rows, cc])
                             * plsc.load_gather(di_v, [rows, cc]))
            out_v[pl.ds(g * _LANES, _LANES)] = acc

        def step(g, _):
            @pl.when(g < n_blocks)
            def _fire():
                fire_block(g)

            @pl.when(g >= _DEPTH - 1)
            def _consume():
                drain_block()
                compute_block(g - (_DEPTH - 1))

            return _

        lax.fori_loop(0, n_blocks + _DEPTH - 1, step, None)

        pltpu.sync_copy(out_v, out_hbm.at[pl.ds(base, bpw)])

    return mf_kernel(users, items, user_embeddings, item_embeddings,
                     user_biases, item_biases)


def kernel(users, items, user_embeddings, item_embeddings, user_biases,
           item_biases):
    out = _mf_forward(users.astype(jnp.int32), items.astype(jnp.int32),
                      user_embeddings, item_embeddings,
                      user_biases.reshape(-1), item_biases.reshape(-1))
    return out.reshape(-1, 1)
